# baseline (speedup 1.0000x reference)
import functools

import jax
import jax.numpy as jnp
from jax import lax
from jax.experimental import pallas as pl
from jax.experimental.pallas import tpu as pltpu
from jax.experimental.pallas import tpu_sc as plsc

N = 10000
E = 320000
D = 128
ALPHA_SLOPE = 0.2
RATE_MIX = 0.5

NW = 32
CHUNK = 80
NCH = 125
NH0 = 64
NH1 = NCH - NH0
ROWS_PER_WRITER = 1000
NWRITERS = N // ROWS_PER_WRITER

_MESH = plsc.VectorSubcoreMesh(core_axis_name="c", subcore_axis_name="s")


@functools.partial(
    pl.kernel,
    mesh=_MESH,
    out_type=jax.ShapeDtypeStruct((2, 2, N, D), jnp.float32),
    scratch_types=[
        pltpu.VMEM((NH0, CHUNK), jnp.int32),
        pltpu.VMEM((NH0, CHUNK), jnp.int32),
        pltpu.VMEM((CHUNK, D), jnp.float32),
        pltpu.VMEM((CHUNK, D), jnp.float32),
        pltpu.VMEM_SHARED((N, D), jnp.float32),
        pltpu.SemaphoreType.DMA,
        pltpu.SemaphoreType.DMA,
    ],
)
def _segsum_sc(t0_hbm, s0_hbm, d0_hbm, t1_hbm, s1_hbm, d1_hbm, out_hbm,
               src_v, dst_v, rows_v, rows1_v, acc_sh, sem, sem1):
    cid = lax.axis_index("c")
    sid = lax.axis_index("s")
    wid = sid * 2 + cid

    zvec = jnp.zeros((16,), jnp.float32)

    def _zrow(r, carry):
        for k in range(D // 16):
            rows_v[r, pl.ds(k * 16, 16)] = zvec
        return carry

    lax.fori_loop(0, CHUNK, _zrow, 0)

    def _zero_acc():
        base = sid * ROWS_PER_WRITER
        for t in range(ROWS_PER_WRITER // CHUNK):
            pltpu.sync_copy(rows_v, acc_sh.at[pl.ds(base + t * CHUNK, CHUNK)])
        pltpu.sync_copy(rows_v.at[pl.ds(0, 40)],
                        acc_sh.at[pl.ds(base + 960, 40)])

    pl.when(sid < NWRITERS)(_zero_acc)
    plsc.subcore_barrier()

    def _wait_rows(buf, s):
        pltpu.make_async_copy(t0_hbm.at[pl.ds(0, CHUNK)], buf, s).wait()

    for b, (tab, srcr, dstr) in enumerate(((t0_hbm, s0_hbm, d0_hbm),
                                           (t1_hbm, s1_hbm, d1_hbm))):
        for h, hn in ((0, NH0), (1, NH1)):
            pltpu.sync_copy(srcr.at[wid, pl.ds(h * NH0, hn)],
                            src_v.at[pl.ds(0, hn)])
            pltpu.sync_copy(dstr.at[wid, pl.ds(h * NH0, hn)],
                            dst_v.at[pl.ds(0, hn)])
            pltpu.async_copy(tab.at[src_v.at[0]], rows_v, sem)

            def _pair(p, carry):
                j0 = 2 * p
                pltpu.async_copy(tab.at[src_v.at[j0 + 1]], rows1_v, sem1)
                _wait_rows(rows_v, sem)
                pltpu.sync_copy(rows_v, acc_sh.at[dst_v.at[j0]], add=True)
                pltpu.async_copy(tab.at[src_v.at[j0 + 2]], rows_v, sem)
                _wait_rows(rows1_v, sem1)
                pltpu.sync_copy(rows1_v, acc_sh.at[dst_v.at[j0 + 1]], add=True)
                return carry

            npairs = (hn - 2) // 2 if hn % 2 == 0 else (hn - 1) // 2
            lax.fori_loop(0, npairs, _pair, 0)
            if hn % 2 == 0:
                pltpu.async_copy(tab.at[src_v.at[hn - 1]], rows1_v, sem1)
                _wait_rows(rows_v, sem)
                pltpu.sync_copy(rows_v, acc_sh.at[dst_v.at[hn - 2]], add=True)
                _wait_rows(rows1_v, sem1)
                pltpu.sync_copy(rows1_v, acc_sh.at[dst_v.at[hn - 1]], add=True)
            else:
                _wait_rows(rows_v, sem)
                pltpu.sync_copy(rows_v, acc_sh.at[dst_v.at[hn - 1]], add=True)
        plsc.subcore_barrier()

        def _drain():
            rows = pl.ds(sid * ROWS_PER_WRITER, ROWS_PER_WRITER)
            pltpu.sync_copy(acc_sh.at[rows], out_hbm.at[b, cid, rows])
            if b == 0:
                lax.fori_loop(0, CHUNK, _zrow, 0)
                _zero_acc()

        pl.when(sid < NWRITERS)(_drain)
        if b == 0:
            plsc.subcore_barrier()


def _segment_sum2(table0, edges0, table1, edges1):
    d0 = edges0[0].reshape(NW, NCH, CHUNK)
    s0 = edges0[1].reshape(NW, NCH, CHUNK)
    d1 = edges1[0].reshape(NW, NCH, CHUNK)
    s1 = edges1[1].reshape(NW, NCH, CHUNK)
    return _segsum_sc(table0, s0, d0, table1, s1, d1)


RB = 2000
NB = N // RB


def _leaky(x):
    return jnp.where(x > 0, x, ALPHA_SLOPE * x)


def _mm_batched_body(x_ref, w_ref, o_ref):
    o_ref[...] = jnp.dot(x_ref[0], w_ref[0],
                         preferred_element_type=jnp.float32)[None]


def _support1(ufeas, gw1):
    return pl.pallas_call(
        _mm_batched_body,
        grid=(2, NB),
        in_specs=[
            pl.BlockSpec((1, RB, D), lambda i, b: (i, b, 0)),
            pl.BlockSpec((1, D, D), lambda i, b: (i, 0, 0)),
        ],
        out_specs=pl.BlockSpec((1, RB, D), lambda i, b: (i, b, 0)),
        out_shape=jax.ShapeDtypeStruct((2, N, D), jnp.float32),
    )(ufeas, gw1)


def _stage_mid_body(p_ref, b_ref, w_ref, o_ref):
    agg = p_ref[0, 0] + p_ref[0, 1]
    h = _leaky(agg + b_ref[0, 0])
    o_ref[...] = jnp.dot(h, w_ref[0], preferred_element_type=jnp.float32)[None]


def _stage_mid(parts, bias, w):
    return pl.pallas_call(
        _stage_mid_body,
        grid=(2, NB),
        in_specs=[
            pl.BlockSpec((1, 2, RB, D), lambda i, bk: (i, 0, bk, 0)),
            pl.BlockSpec((1, 1, D), lambda i, bk: (i, 0, 0)),
            pl.BlockSpec((1, D, D), lambda i, bk: (i, 0, 0)),
        ],
        out_specs=pl.BlockSpec((1, RB, D), lambda i, bk: (i, bk, 0)),
        out_shape=jax.ShapeDtypeStruct((2, N, D), jnp.float32),
    )(parts, bias.reshape(2, 1, D), w)


def _head_body(p_ref, gb_ref, uf_ref, wa_ref, wb_ref, ub_ref, o_ref):
    acc = None
    for i in range(2):
        h = _leaky(p_ref[i, 0] + p_ref[i, 1] + gb_ref[i, 0])
        out = (jnp.dot(h, wa_ref[i], preferred_element_type=jnp.float32)
               + jnp.dot(uf_ref[i], wb_ref[i],
                         preferred_element_type=jnp.float32)
               + ub_ref[i, 0])
        r = jnp.maximum(out, 0.0)
        acc = RATE_MIX * r if acc is None else acc + (1.0 - RATE_MIX) * r
    o_ref[...] = acc


def _head(parts, gb2, ufeas, uwa, uwb, ub):
    return pl.pallas_call(
        _head_body,
        grid=(NB,),
        in_specs=[
            pl.BlockSpec((2, 2, RB, D), lambda bk: (0, 0, bk, 0)),
            pl.BlockSpec((2, 1, D), lambda bk: (0, 0, 0)),
            pl.BlockSpec((2, RB, D), lambda bk: (0, bk, 0)),
            pl.BlockSpec((2, D, D), lambda bk: (0, 0, 0)),
            pl.BlockSpec((2, D, D), lambda bk: (0, 0, 0)),
            pl.BlockSpec((2, 1, D), lambda bk: (0, 0, 0)),
        ],
        out_specs=pl.BlockSpec((RB, D), lambda bk: (bk, 0)),
        out_shape=jax.ShapeDtypeStruct((N, D), jnp.float32),
    )(parts, gb2.reshape(2, 1, D), ufeas, uwa, uwb, ub.reshape(2, 1, D))


def kernel(UFEAs, UVs, VUs, gw1, gb1, gw2, gb2, uw, ub):
    support1 = _support1(UFEAs, gw1)
    p1 = _segment_sum2(support1[0], VUs[0], support1[1], VUs[1])
    support2 = _stage_mid(p1, gb1, gw2)
    p2 = _segment_sum2(support2[0], UVs[0], support2[1], UVs[1])
    return _head(p2, gb2, UFEAs, uw[:, :D], uw[:, D:], ub)


# speedup vs baseline: 1.0434x; 1.0434x over previous
import functools

import jax
import jax.numpy as jnp
from jax import lax
from jax.experimental import pallas as pl
from jax.experimental.pallas import tpu as pltpu
from jax.experimental.pallas import tpu_sc as plsc

N = 10000
E = 320000
D = 128
ALPHA_SLOPE = 0.2
RATE_MIX = 0.5

NW = 32
CHUNK = 80
NCH = 125
NH0 = 64
NH1 = NCH - NH0
WR = 624
WR_LAST = N - 15 * WR

_MESH = plsc.VectorSubcoreMesh(core_axis_name="c", subcore_axis_name="s")


@functools.partial(
    pl.kernel,
    mesh=_MESH,
    out_type=jax.ShapeDtypeStruct((2, 2, N, D), jnp.float32),
    scratch_types=[
        pltpu.VMEM((NH0, CHUNK), jnp.int32),
        pltpu.VMEM((NH0, CHUNK), jnp.int32),
        pltpu.VMEM((CHUNK, D), jnp.float32),
        pltpu.VMEM((CHUNK, D), jnp.float32),
        pltpu.VMEM_SHARED((N, D), jnp.float32),
        pltpu.SemaphoreType.DMA,
        pltpu.SemaphoreType.DMA,
    ],
)
def _segsum_sc(tabs_hbm, s0_hbm, d0_hbm, s1_hbm, d1_hbm, out_hbm,
               src_v, dst_v, rows_v, rows1_v, acc_sh, sem, sem1):
    cid = lax.axis_index("c")
    sid = lax.axis_index("s")
    wid = sid * 2 + cid

    zvec = jnp.zeros((16,), jnp.float32)

    def _zrow(r, carry):
        for k in range(D // 16):
            rows_v[r, pl.ds(k * 16, 16)] = zvec
        return carry

    def _zero_acc():
        base = sid * WR

        @pl.when(sid < 15)
        def _z_main():
            for t in range(WR // CHUNK):
                pltpu.sync_copy(rows_v,
                                acc_sh.at[pl.ds(base + t * CHUNK, CHUNK)])
            pltpu.sync_copy(rows_v.at[pl.ds(0, WR % CHUNK)],
                            acc_sh.at[pl.ds(base + WR - WR % CHUNK,
                                            WR % CHUNK)])

        @pl.when(sid == 15)
        def _z_last():
            for t in range(WR_LAST // CHUNK):
                pltpu.sync_copy(rows_v,
                                acc_sh.at[pl.ds(base + t * CHUNK, CHUNK)])

    def _write_out(b):
        @pl.when(sid < 15)
        def _w_main():
            rows = pl.ds(sid * WR, WR)
            pltpu.sync_copy(acc_sh.at[rows], out_hbm.at[b, cid, rows])

        @pl.when(sid == 15)
        def _w_last():
            rows = pl.ds(15 * WR, WR_LAST)
            pltpu.sync_copy(acc_sh.at[rows], out_hbm.at[b, cid, rows])

    lax.fori_loop(0, CHUNK, _zrow, 0)

    pltpu.sync_copy(s0_hbm.at[wid, pl.ds(0, NH0)], src_v)
    pltpu.sync_copy(d0_hbm.at[wid, pl.ds(0, NH0)], dst_v)
    _zero_acc()
    pltpu.async_copy(tabs_hbm.at[0].at[src_v.at[0]], rows_v, sem)
    plsc.subcore_barrier()

    def _wait_rows(buf, s):
        pltpu.make_async_copy(tabs_hbm.at[0].at[pl.ds(0, CHUNK)],
                              buf, s).wait()

    for b, (srcr, dstr) in enumerate(((s0_hbm, d0_hbm), (s1_hbm, d1_hbm))):
        tab = tabs_hbm.at[b]
        for h, hn in ((0, NH0), (1, NH1)):
            if h != 0:
                pltpu.sync_copy(srcr.at[wid, pl.ds(h * NH0, hn)],
                                src_v.at[pl.ds(0, hn)])
                pltpu.sync_copy(dstr.at[wid, pl.ds(h * NH0, hn)],
                                dst_v.at[pl.ds(0, hn)])
                pltpu.async_copy(tab.at[src_v.at[0]], rows_v, sem)

            def _pair(p, carry):
                j0 = 2 * p
                pltpu.async_copy(tab.at[src_v.at[j0 + 1]], rows1_v, sem1)
                _wait_rows(rows_v, sem)
                pltpu.sync_copy(rows_v, acc_sh.at[dst_v.at[j0]], add=True)
                pltpu.async_copy(tab.at[src_v.at[j0 + 2]], rows_v, sem)
                _wait_rows(rows1_v, sem1)
                pltpu.sync_copy(rows1_v, acc_sh.at[dst_v.at[j0 + 1]], add=True)
                return carry

            npairs = (hn - 2) // 2 if hn % 2 == 0 else (hn - 1) // 2
            lax.fori_loop(0, npairs, _pair, 0)
            if hn % 2 == 0:
                pltpu.async_copy(tab.at[src_v.at[hn - 1]], rows1_v, sem1)
                _wait_rows(rows_v, sem)
                pltpu.sync_copy(rows_v, acc_sh.at[dst_v.at[hn - 2]], add=True)
                _wait_rows(rows1_v, sem1)
                pltpu.sync_copy(rows1_v, acc_sh.at[dst_v.at[hn - 1]], add=True)
            else:
                _wait_rows(rows_v, sem)
                pltpu.sync_copy(rows_v, acc_sh.at[dst_v.at[hn - 1]], add=True)
        plsc.subcore_barrier()

        _write_out(b)
        if b == 0:
            lax.fori_loop(0, CHUNK, _zrow, 0)
            pltpu.sync_copy(s1_hbm.at[wid, pl.ds(0, NH0)], src_v)
            pltpu.sync_copy(d1_hbm.at[wid, pl.ds(0, NH0)], dst_v)
            _zero_acc()
            pltpu.async_copy(tabs_hbm.at[1].at[src_v.at[0]], rows_v, sem)
            plsc.subcore_barrier()


def _segment_sum2(tables, edges0, edges1):
    d0 = edges0[0].reshape(NW, NCH, CHUNK)
    s0 = edges0[1].reshape(NW, NCH, CHUNK)
    d1 = edges1[0].reshape(NW, NCH, CHUNK)
    s1 = edges1[1].reshape(NW, NCH, CHUNK)
    return _segsum_sc(tables, s0, d0, s1, d1)


RB = 2000
NB = N // RB


def _leaky(x):
    return jnp.where(x > 0, x, ALPHA_SLOPE * x)


def _mm_batched_body(x_ref, w_ref, o_ref):
    o_ref[...] = jnp.dot(x_ref[0], w_ref[0],
                         preferred_element_type=jnp.float32)[None]


def _support1(ufeas, gw1):
    return pl.pallas_call(
        _mm_batched_body,
        grid=(2, NB),
        in_specs=[
            pl.BlockSpec((1, RB, D), lambda i, b: (i, b, 0)),
            pl.BlockSpec((1, D, D), lambda i, b: (i, 0, 0)),
        ],
        out_specs=pl.BlockSpec((1, RB, D), lambda i, b: (i, b, 0)),
        out_shape=jax.ShapeDtypeStruct((2, N, D), jnp.float32),
    )(ufeas, gw1)


def _stage_mid_body(p_ref, b_ref, w_ref, o_ref):
    agg = p_ref[0, 0] + p_ref[0, 1]
    h = _leaky(agg + b_ref[0, 0])
    o_ref[...] = jnp.dot(h, w_ref[0], preferred_element_type=jnp.float32)[None]


def _stage_mid(parts, bias, w):
    return pl.pallas_call(
        _stage_mid_body,
        grid=(2, NB),
        in_specs=[
            pl.BlockSpec((1, 2, RB, D), lambda i, bk: (i, 0, bk, 0)),
            pl.BlockSpec((1, 1, D), lambda i, bk: (i, 0, 0)),
            pl.BlockSpec((1, D, D), lambda i, bk: (i, 0, 0)),
        ],
        out_specs=pl.BlockSpec((1, RB, D), lambda i, bk: (i, bk, 0)),
        out_shape=jax.ShapeDtypeStruct((2, N, D), jnp.float32),
    )(parts, bias.reshape(2, 1, D), w)


def _head_body(p_ref, gb_ref, uf_ref, wa_ref, wb_ref, ub_ref, o_ref):
    acc = None
    for i in range(2):
        h = _leaky(p_ref[i, 0] + p_ref[i, 1] + gb_ref[i, 0])
        out = (jnp.dot(h, wa_ref[i], preferred_element_type=jnp.float32)
               + jnp.dot(uf_ref[i], wb_ref[i],
                         preferred_element_type=jnp.float32)
               + ub_ref[i, 0])
        r = jnp.maximum(out, 0.0)
        acc = RATE_MIX * r if acc is None else acc + (1.0 - RATE_MIX) * r
    o_ref[...] = acc


def _head(parts, gb2, ufeas, uwa, uwb, ub):
    return pl.pallas_call(
        _head_body,
        grid=(NB,),
        in_specs=[
            pl.BlockSpec((2, 2, RB, D), lambda bk: (0, 0, bk, 0)),
            pl.BlockSpec((2, 1, D), lambda bk: (0, 0, 0)),
            pl.BlockSpec((2, RB, D), lambda bk: (0, bk, 0)),
            pl.BlockSpec((2, D, D), lambda bk: (0, 0, 0)),
            pl.BlockSpec((2, D, D), lambda bk: (0, 0, 0)),
            pl.BlockSpec((2, 1, D), lambda bk: (0, 0, 0)),
        ],
        out_specs=pl.BlockSpec((RB, D), lambda bk: (bk, 0)),
        out_shape=jax.ShapeDtypeStruct((N, D), jnp.float32),
    )(parts, gb2.reshape(2, 1, D), ufeas, uwa, uwb, ub.reshape(2, 1, D))


def kernel(UFEAs, UVs, VUs, gw1, gb1, gw2, gb2, uw, ub):
    support1 = _support1(UFEAs, gw1)
    p1 = _segment_sum2(support1, VUs[0], VUs[1])
    support2 = _stage_mid(p1, gb1, gw2)
    p2 = _segment_sum2(support2, UVs[0], UVs[1])
    return _head(p2, gb2, UFEAs, uw[:, :D], uw[:, D:], ub)


# speedup vs baseline: 1.0453x; 1.0018x over previous
import functools

import jax
import jax.numpy as jnp
from jax import lax
from jax.experimental import pallas as pl
from jax.experimental.pallas import tpu as pltpu
from jax.experimental.pallas import tpu_sc as plsc

N = 10000
E = 320000
D = 128
ALPHA_SLOPE = 0.2
RATE_MIX = 0.5

NW = 32
CHUNK = 80
NCH = 125
BLK = 40
BLOCKS = (BLK, BLK, BLK, NCH - 3 * BLK)
WR = 624
WR_LAST = N - 15 * WR

_MESH = plsc.VectorSubcoreMesh(core_axis_name="c", subcore_axis_name="s")


@functools.partial(
    pl.kernel,
    mesh=_MESH,
    out_type=jax.ShapeDtypeStruct((2, 2, N, D), jnp.float32),
    scratch_types=[
        pltpu.VMEM((BLK, CHUNK), jnp.int32),
        pltpu.VMEM((BLK, CHUNK), jnp.int32),
        pltpu.VMEM((CHUNK, D), jnp.float32),
        pltpu.VMEM((CHUNK, D), jnp.float32),
        pltpu.VMEM((CHUNK, D), jnp.float32),
        pltpu.VMEM_SHARED((N, D), jnp.float32),
        pltpu.SemaphoreType.DMA,
        pltpu.SemaphoreType.DMA,
        pltpu.SemaphoreType.DMA,
        pltpu.SemaphoreType.DMA,
        pltpu.SemaphoreType.DMA,
        pltpu.SemaphoreType.DMA,
    ],
)
def _segsum_sc(tabs_hbm, s0_hbm, d0_hbm, s1_hbm, d1_hbm, out_hbm,
               src_v, dst_v, rows_v, rows1_v, rows2_v, acc_sh,
               sg0, sg1, sg2, ss0, ss1, ss2):
    cid = lax.axis_index("c")
    sid = lax.axis_index("s")
    wid = sid * 2 + cid

    zvec = jnp.zeros((16,), jnp.float32)

    def _zrow(r, carry):
        for k in range(D // 16):
            rows_v[r, pl.ds(k * 16, 16)] = zvec
        return carry

    def _zero_acc():
        base = sid * WR

        @pl.when(sid < 15)
        def _z_main():
            for t in range(WR // CHUNK):
                pltpu.sync_copy(rows_v,
                                acc_sh.at[pl.ds(base + t * CHUNK, CHUNK)])
            pltpu.sync_copy(rows_v.at[pl.ds(0, WR % CHUNK)],
                            acc_sh.at[pl.ds(base + WR - WR % CHUNK,
                                            WR % CHUNK)])

        @pl.when(sid == 15)
        def _z_last():
            for t in range(WR_LAST // CHUNK):
                pltpu.sync_copy(rows_v,
                                acc_sh.at[pl.ds(base + t * CHUNK, CHUNK)])

    def _write_out(b):
        @pl.when(sid < 15)
        def _w_main():
            rows = pl.ds(sid * WR, WR)
            pltpu.sync_copy(acc_sh.at[rows], out_hbm.at[b, cid, rows])

        @pl.when(sid == 15)
        def _w_last():
            rows = pl.ds(15 * WR, WR_LAST)
            pltpu.sync_copy(acc_sh.at[rows], out_hbm.at[b, cid, rows])

    bufs = (rows_v, rows1_v, rows2_v)
    sgs = (sg0, sg1, sg2)
    sss = (ss0, ss1, ss2)

    def _wait(slot, sems):
        pltpu.make_async_copy(tabs_hbm.at[0].at[pl.ds(0, CHUNK)],
                              bufs[slot], sems[slot]).wait()

    def _issue_g(tab, j, slot):
        pltpu.async_copy(tab.at[src_v.at[j]], bufs[slot], sgs[slot])

    def _issue_s(j, slot):
        pltpu.async_copy(bufs[slot], acc_sh.at[dst_v.at[j]], sss[slot],
                         add=True)

    def _run_block(tab, hn):
        _issue_g(tab, 0, 0)
        _issue_g(tab, 1, 1)
        _wait(0, sgs); _issue_s(0, 0)
        if hn > 2:
            _issue_g(tab, 2, 2)
        _wait(1, sgs); _issue_s(1, 1)
        if hn > 3:
            _wait(0, sss)
            _issue_g(tab, 3, 0)

        def _step(j, slot):
            _wait(slot, sgs)
            _issue_s(j, slot)
            _wait((slot + 2) % 3, sss)
            _issue_g(tab, j + 2, (slot + 2) % 3)

        ntrip = (hn - 4) // 3 if hn > 4 else 0
        if ntrip > 0:
            def _triple(m, carry):
                j0 = 2 + 3 * m
                _step(j0, 2)
                _step(j0 + 1, 0)
                _step(j0 + 2, 1)
                return carry

            lax.fori_loop(0, ntrip, _triple, 0)
        for j in range(2 + 3 * ntrip, hn - 2):
            _step(j, j % 3)
        for j in (hn - 2, hn - 1):
            if j < 2:
                continue
            slot = j % 3
            _wait(slot, sgs)
            _issue_s(j, slot)
            _wait((slot + 2) % 3, sss)
        _wait((hn - 1) % 3, sss)

    def _load_idx(srcr, dstr, blk_i, hn):
        pltpu.sync_copy(srcr.at[wid, pl.ds(blk_i * BLK, hn)],
                        src_v.at[pl.ds(0, hn)])
        pltpu.sync_copy(dstr.at[wid, pl.ds(blk_i * BLK, hn)],
                        dst_v.at[pl.ds(0, hn)])

    lax.fori_loop(0, CHUNK, _zrow, 0)
    _load_idx(s0_hbm, d0_hbm, 0, BLK)
    _zero_acc()
    plsc.subcore_barrier()

    for b, (srcr, dstr) in enumerate(((s0_hbm, d0_hbm), (s1_hbm, d1_hbm))):
        tab = tabs_hbm.at[b]
        for blk_i, hn in enumerate(BLOCKS):
            if not (blk_i == 0 and b == 0):
                _load_idx(srcr, dstr, blk_i, hn)
            _run_block(tab, hn)
        plsc.subcore_barrier()

        _write_out(b)
        if b == 0:
            lax.fori_loop(0, CHUNK, _zrow, 0)
            _zero_acc()
            _load_idx(s1_hbm, d1_hbm, 0, BLK)
            plsc.subcore_barrier()


def _segment_sum2(tables, edges0, edges1):
    d0 = edges0[0].reshape(NW, NCH, CHUNK)
    s0 = edges0[1].reshape(NW, NCH, CHUNK)
    d1 = edges1[0].reshape(NW, NCH, CHUNK)
    s1 = edges1[1].reshape(NW, NCH, CHUNK)
    return _segsum_sc(tables, s0, d0, s1, d1)


RB = 2000
NB = N // RB


def _leaky(x):
    return jnp.where(x > 0, x, ALPHA_SLOPE * x)


def _mm_batched_body(x_ref, w_ref, o_ref):
    o_ref[...] = jnp.dot(x_ref[0], w_ref[0],
                         preferred_element_type=jnp.float32)[None]


def _support1(ufeas, gw1):
    return pl.pallas_call(
        _mm_batched_body,
        grid=(2, NB),
        in_specs=[
            pl.BlockSpec((1, RB, D), lambda i, b: (i, b, 0)),
            pl.BlockSpec((1, D, D), lambda i, b: (i, 0, 0)),
        ],
        out_specs=pl.BlockSpec((1, RB, D), lambda i, b: (i, b, 0)),
        out_shape=jax.ShapeDtypeStruct((2, N, D), jnp.float32),
    )(ufeas, gw1)


def _stage_mid_body(p_ref, b_ref, w_ref, o_ref):
    agg = p_ref[0, 0] + p_ref[0, 1]
    h = _leaky(agg + b_ref[0, 0])
    o_ref[...] = jnp.dot(h, w_ref[0], preferred_element_type=jnp.float32)[None]


def _stage_mid(parts, bias, w):
    return pl.pallas_call(
        _stage_mid_body,
        grid=(2, NB),
        in_specs=[
            pl.BlockSpec((1, 2, RB, D), lambda i, bk: (i, 0, bk, 0)),
            pl.BlockSpec((1, 1, D), lambda i, bk: (i, 0, 0)),
            pl.BlockSpec((1, D, D), lambda i, bk: (i, 0, 0)),
        ],
        out_specs=pl.BlockSpec((1, RB, D), lambda i, bk: (i, bk, 0)),
        out_shape=jax.ShapeDtypeStruct((2, N, D), jnp.float32),
    )(parts, bias.reshape(2, 1, D), w)


def _head_body(p_ref, gb_ref, uf_ref, wa_ref, wb_ref, ub_ref, o_ref):
    acc = None
    for i in range(2):
        h = _leaky(p_ref[i, 0] + p_ref[i, 1] + gb_ref[i, 0])
        out = (jnp.dot(h, wa_ref[i], preferred_element_type=jnp.float32)
               + jnp.dot(uf_ref[i], wb_ref[i],
                         preferred_element_type=jnp.float32)
               + ub_ref[i, 0])
        r = jnp.maximum(out, 0.0)
        acc = RATE_MIX * r if acc is None else acc + (1.0 - RATE_MIX) * r
    o_ref[...] = acc


def _head(parts, gb2, ufeas, uwa, uwb, ub):
    return pl.pallas_call(
        _head_body,
        grid=(NB,),
        in_specs=[
            pl.BlockSpec((2, 2, RB, D), lambda bk: (0, 0, bk, 0)),
            pl.BlockSpec((2, 1, D), lambda bk: (0, 0, 0)),
            pl.BlockSpec((2, RB, D), lambda bk: (0, bk, 0)),
            pl.BlockSpec((2, D, D), lambda bk: (0, 0, 0)),
            pl.BlockSpec((2, D, D), lambda bk: (0, 0, 0)),
            pl.BlockSpec((2, 1, D), lambda bk: (0, 0, 0)),
        ],
        out_specs=pl.BlockSpec((RB, D), lambda bk: (bk, 0)),
        out_shape=jax.ShapeDtypeStruct((N, D), jnp.float32),
    )(parts, gb2.reshape(2, 1, D), ufeas, uwa, uwb, ub.reshape(2, 1, D))


def kernel(UFEAs, UVs, VUs, gw1, gb1, gw2, gb2, uw, ub):
    support1 = _support1(UFEAs, gw1)
    p1 = _segment_sum2(support1, VUs[0], VUs[1])
    support2 = _stage_mid(p1, gb1, gw2)
    p2 = _segment_sum2(support2, UVs[0], UVs[1])
    return _head(p2, gb2, UFEAs, uw[:, :D], uw[:, D:], ub)


# speedup vs baseline: 1.1924x; 1.1408x over previous
import functools

import jax
import jax.numpy as jnp
from jax import lax
from jax.experimental import pallas as pl
from jax.experimental.pallas import tpu as pltpu
from jax.experimental.pallas import tpu_sc as plsc

N = 10000
E = 320000
D = 128
ALPHA_SLOPE = 0.2
RATE_MIX = 0.5

NW = 32
CHUNK = 80
NCH = 125
BLK = 40
BLOCKS = (BLK, BLK, BLK, NCH - 3 * BLK)
WR = 624
WR_LAST = N - 15 * WR

_MESH = plsc.VectorSubcoreMesh(core_axis_name="c", subcore_axis_name="s")


@functools.partial(
    pl.kernel,
    mesh=_MESH,
    out_type=jax.ShapeDtypeStruct((2, 2, N, D), jnp.float32),
    scratch_types=[
        pltpu.VMEM((BLK, CHUNK), jnp.int32),
        pltpu.VMEM((BLK, CHUNK), jnp.int32),
        pltpu.VMEM((CHUNK, D), jnp.float32),
        pltpu.VMEM((CHUNK, D), jnp.float32),
        pltpu.VMEM((CHUNK, D), jnp.float32),
        pltpu.VMEM_SHARED((N, D), jnp.float32),
        pltpu.SemaphoreType.DMA,
        pltpu.SemaphoreType.DMA,
        pltpu.SemaphoreType.DMA,
        pltpu.SemaphoreType.DMA,
        pltpu.SemaphoreType.DMA,
        pltpu.SemaphoreType.DMA,
        pltpu.SemaphoreType.DMA,
    ],
)
def _segsum_sc(tabs_hbm, s0_hbm, d0_hbm, s1_hbm, d1_hbm, out_hbm,
               src_v, dst_v, rows_v, rows1_v, rows2_v, acc_sh,
               sg0, sg1, sg2, ss0, ss1, ss2, sidx):
    cid = lax.axis_index("c")
    sid = lax.axis_index("s")
    wid = sid * 2 + cid

    zvec = jnp.zeros((16,), jnp.float32)

    def _zrow(r, carry):
        for k in range(D // 16):
            rows_v[r, pl.ds(k * 16, 16)] = zvec
        return carry

    def _zero_acc():
        base = sid * WR

        @pl.when(sid < 15)
        def _z_main():
            for t in range(WR // CHUNK):
                pltpu.sync_copy(rows_v,
                                acc_sh.at[pl.ds(base + t * CHUNK, CHUNK)])
            pltpu.sync_copy(rows_v.at[pl.ds(0, WR % CHUNK)],
                            acc_sh.at[pl.ds(base + WR - WR % CHUNK,
                                            WR % CHUNK)])

        @pl.when(sid == 15)
        def _z_last():
            for t in range(WR_LAST // CHUNK):
                pltpu.sync_copy(rows_v,
                                acc_sh.at[pl.ds(base + t * CHUNK, CHUNK)])

    def _write_out(b):
        @pl.when(sid < 15)
        def _w_main():
            rows = pl.ds(sid * WR, WR)
            pltpu.sync_copy(acc_sh.at[rows], out_hbm.at[b, cid, rows])

        @pl.when(sid == 15)
        def _w_last():
            rows = pl.ds(15 * WR, WR_LAST)
            pltpu.sync_copy(acc_sh.at[rows], out_hbm.at[b, cid, rows])

    bufs = (rows_v, rows1_v, rows2_v)
    sgs = (sg0, sg1, sg2)
    sss = (ss0, ss1, ss2)

    def _wait(slot, sems):
        pltpu.make_async_copy(tabs_hbm.at[0].at[pl.ds(0, CHUNK)],
                              bufs[slot], sems[slot]).wait()

    def _issue_g(tab, j, slot):
        pltpu.async_copy(tab.at[src_v.at[j]], bufs[slot], sgs[slot])

    def _issue_s(j, slot):
        pltpu.async_copy(bufs[slot], acc_sh.at[dst_v.at[j]], sss[slot],
                         add=True)

    def _run_block(tab, hn, prefetch=None):
        _issue_g(tab, 0, 0)
        _issue_g(tab, 1, 1)
        _issue_g(tab, 2, 2)
        _wait(0, sgs)
        _issue_s(0, 0)

        def _stepj(j, slot):
            _wait(slot, sss)
            _issue_g(tab, j, slot)
            gslot = (slot + 1) % 3
            _wait(gslot, sgs)
            _issue_s(j - 2, gslot)

        ntrip = (hn - 3) // 3
        if ntrip > 0:
            def _triple(m, carry):
                j0 = 3 + 3 * m
                _stepj(j0, 0)
                _stepj(j0 + 1, 1)
                _stepj(j0 + 2, 2)
                return carry

            lax.fori_loop(0, ntrip, _triple, 0)
        for j in range(3 + 3 * ntrip, hn):
            _stepj(j, j % 3)
        _wait(hn % 3, sss)
        _wait((hn + 1) % 3, sgs)
        _issue_s(hn - 2, (hn + 1) % 3)
        _wait((hn + 1) % 3, sss)
        _wait((hn + 2) % 3, sgs)
        _issue_s(hn - 1, (hn + 2) % 3)
        if prefetch is not None:
            prefetch[0]()
        _wait((hn + 2) % 3, sss)
        if prefetch is not None:
            prefetch[1]()

    def _idx_prefetch_src(srcr, blk_i, hn):
        pltpu.async_copy(srcr.at[wid, pl.ds(blk_i * BLK, hn)],
                         src_v.at[pl.ds(0, hn)], sidx)

    def _idx_prefetch_dst(dstr, blk_i, hn):
        pltpu.async_copy(dstr.at[wid, pl.ds(blk_i * BLK, hn)],
                         dst_v.at[pl.ds(0, hn)], sidx)

    def _idx_wait(hn):
        for ref in (src_v, dst_v):
            pltpu.make_async_copy(s0_hbm.at[0, pl.ds(0, hn)],
                                  ref.at[pl.ds(0, hn)], sidx).wait()

    def _load_idx(srcr, dstr, blk_i, hn):
        pltpu.sync_copy(srcr.at[wid, pl.ds(blk_i * BLK, hn)],
                        src_v.at[pl.ds(0, hn)])
        pltpu.sync_copy(dstr.at[wid, pl.ds(blk_i * BLK, hn)],
                        dst_v.at[pl.ds(0, hn)])

    lax.fori_loop(0, CHUNK, _zrow, 0)
    _load_idx(s0_hbm, d0_hbm, 0, BLK)
    _zero_acc()
    plsc.subcore_barrier()

    nblocks = len(BLOCKS)
    for b, (srcr, dstr) in enumerate(((s0_hbm, d0_hbm), (s1_hbm, d1_hbm))):
        tab = tabs_hbm.at[b]
        for blk_i, hn in enumerate(BLOCKS):
            if not (blk_i == 0 and b == 0):
                _idx_wait(hn)
            if blk_i + 1 < nblocks:
                nxt = BLOCKS[blk_i + 1]
                pf = ((lambda bi=blk_i + 1, nh=nxt:
                       _idx_prefetch_src(srcr, bi, nh)),
                      (lambda bi=blk_i + 1, nh=nxt:
                       _idx_prefetch_dst(dstr, bi, nh)))
            elif b == 0:
                pf = ((lambda: _idx_prefetch_src(s1_hbm, 0, BLOCKS[0])),
                      (lambda: _idx_prefetch_dst(d1_hbm, 0, BLOCKS[0])))
            else:
                pf = None
            _run_block(tab, hn, prefetch=pf)
        plsc.subcore_barrier()

        _write_out(b)
        if b == 0:
            lax.fori_loop(0, CHUNK, _zrow, 0)
            _zero_acc()
            plsc.subcore_barrier()


def _segment_sum2(tables, edges0, edges1):
    d0 = edges0[0].reshape(NW, NCH, CHUNK)
    s0 = edges0[1].reshape(NW, NCH, CHUNK)
    d1 = edges1[0].reshape(NW, NCH, CHUNK)
    s1 = edges1[1].reshape(NW, NCH, CHUNK)
    return _segsum_sc(tables, s0, d0, s1, d1)


RB = 2000
NB = N // RB


def _leaky(x):
    return jnp.where(x > 0, x, ALPHA_SLOPE * x)


def _mm_batched_body(x_ref, w_ref, o_ref):
    o_ref[...] = jnp.dot(x_ref[0], w_ref[0],
                         preferred_element_type=jnp.float32)[None]


def _support1(ufeas, gw1):
    return pl.pallas_call(
        _mm_batched_body,
        grid=(2, NB),
        in_specs=[
            pl.BlockSpec((1, RB, D), lambda i, b: (i, b, 0)),
            pl.BlockSpec((1, D, D), lambda i, b: (i, 0, 0)),
        ],
        out_specs=pl.BlockSpec((1, RB, D), lambda i, b: (i, b, 0)),
        out_shape=jax.ShapeDtypeStruct((2, N, D), jnp.float32),
    )(ufeas, gw1)


def _stage_mid_body(p_ref, b_ref, w_ref, o_ref):
    agg = p_ref[0, 0] + p_ref[0, 1]
    h = _leaky(agg + b_ref[0, 0])
    o_ref[...] = jnp.dot(h, w_ref[0], preferred_element_type=jnp.float32)[None]


def _stage_mid(parts, bias, w):
    return pl.pallas_call(
        _stage_mid_body,
        grid=(2, NB),
        in_specs=[
            pl.BlockSpec((1, 2, RB, D), lambda i, bk: (i, 0, bk, 0)),
            pl.BlockSpec((1, 1, D), lambda i, bk: (i, 0, 0)),
            pl.BlockSpec((1, D, D), lambda i, bk: (i, 0, 0)),
        ],
        out_specs=pl.BlockSpec((1, RB, D), lambda i, bk: (i, bk, 0)),
        out_shape=jax.ShapeDtypeStruct((2, N, D), jnp.float32),
    )(parts, bias.reshape(2, 1, D), w)


def _head_body(p_ref, gb_ref, uf_ref, wa_ref, wb_ref, ub_ref, o_ref):
    acc = None
    for i in range(2):
        h = _leaky(p_ref[i, 0] + p_ref[i, 1] + gb_ref[i, 0])
        out = (jnp.dot(h, wa_ref[i], preferred_element_type=jnp.float32)
               + jnp.dot(uf_ref[i], wb_ref[i],
                         preferred_element_type=jnp.float32)
               + ub_ref[i, 0])
        r = jnp.maximum(out, 0.0)
        acc = RATE_MIX * r if acc is None else acc + (1.0 - RATE_MIX) * r
    o_ref[...] = acc


def _head(parts, gb2, ufeas, uwa, uwb, ub):
    return pl.pallas_call(
        _head_body,
        grid=(NB,),
        in_specs=[
            pl.BlockSpec((2, 2, RB, D), lambda bk: (0, 0, bk, 0)),
            pl.BlockSpec((2, 1, D), lambda bk: (0, 0, 0)),
            pl.BlockSpec((2, RB, D), lambda bk: (0, bk, 0)),
            pl.BlockSpec((2, D, D), lambda bk: (0, 0, 0)),
            pl.BlockSpec((2, D, D), lambda bk: (0, 0, 0)),
            pl.BlockSpec((2, 1, D), lambda bk: (0, 0, 0)),
        ],
        out_specs=pl.BlockSpec((RB, D), lambda bk: (bk, 0)),
        out_shape=jax.ShapeDtypeStruct((N, D), jnp.float32),
    )(parts, gb2.reshape(2, 1, D), ufeas, uwa, uwb, ub.reshape(2, 1, D))


def kernel(UFEAs, UVs, VUs, gw1, gb1, gw2, gb2, uw, ub):
    support1 = _support1(UFEAs, gw1)
    p1 = _segment_sum2(support1, VUs[0], VUs[1])
    support2 = _stage_mid(p1, gb1, gw2)
    p2 = _segment_sum2(support2, UVs[0], UVs[1])
    return _head(p2, gb2, UFEAs, uw[:, :D], uw[:, D:], ub)


# speedup vs baseline: 1.2047x; 1.0103x over previous
import functools

import jax
import jax.numpy as jnp
from jax import lax
from jax.experimental import pallas as pl
from jax.experimental.pallas import tpu as pltpu
from jax.experimental.pallas import tpu_sc as plsc

N = 10000
E = 320000
D = 128
ALPHA_SLOPE = 0.2
RATE_MIX = 0.5

NW = 32
CHUNK = 80
NCH = 125
BLK = 40
BLOCKS = (BLK, BLK, BLK, NCH - 3 * BLK)
BLK_MAX = max(BLOCKS)
WR = 624
WR_LAST = N - 15 * WR

_MESH = plsc.VectorSubcoreMesh(core_axis_name="c", subcore_axis_name="s")


@functools.partial(
    pl.kernel,
    mesh=_MESH,
    out_type=jax.ShapeDtypeStruct((2, 2, N, D), jnp.float32),
    scratch_types=[
        pltpu.VMEM((BLK_MAX, CHUNK), jnp.int32),
        pltpu.VMEM((BLK_MAX, CHUNK), jnp.int32),
        pltpu.VMEM((CHUNK, D), jnp.float32),
        pltpu.VMEM((CHUNK, D), jnp.float32),
        pltpu.VMEM((CHUNK, D), jnp.float32),
        pltpu.VMEM_SHARED((N, D), jnp.float32),
        pltpu.SemaphoreType.DMA,
        pltpu.SemaphoreType.DMA,
        pltpu.SemaphoreType.DMA,
        pltpu.SemaphoreType.DMA,
        pltpu.SemaphoreType.DMA,
        pltpu.SemaphoreType.DMA,
        pltpu.SemaphoreType.DMA,
    ],
)
def _segsum_sc(tabs_hbm, s0_hbm, d0_hbm, s1_hbm, d1_hbm, out_hbm,
               src_v, dst_v, rows_v, rows1_v, rows2_v, acc_sh,
               sg0, sg1, sg2, ss0, ss1, ss2, sidx):
    cid = lax.axis_index("c")
    sid = lax.axis_index("s")
    wid = sid * 2 + cid

    zvec = jnp.zeros((16,), jnp.float32)

    def _zrow(r, carry):
        for k in range(D // 16):
            rows_v[r, pl.ds(k * 16, 16)] = zvec
        return carry

    def _zero_acc():
        base = sid * WR

        @pl.when(sid < 15)
        def _z_main():
            for t in range(WR // CHUNK):
                pltpu.sync_copy(rows_v,
                                acc_sh.at[pl.ds(base + t * CHUNK, CHUNK)])
            pltpu.sync_copy(rows_v.at[pl.ds(0, WR % CHUNK)],
                            acc_sh.at[pl.ds(base + WR - WR % CHUNK,
                                            WR % CHUNK)])

        @pl.when(sid == 15)
        def _z_last():
            for t in range(WR_LAST // CHUNK):
                pltpu.sync_copy(rows_v,
                                acc_sh.at[pl.ds(base + t * CHUNK, CHUNK)])

    def _write_out(b):
        @pl.when(sid < 15)
        def _w_main():
            rows = pl.ds(sid * WR, WR)
            pltpu.sync_copy(acc_sh.at[rows], out_hbm.at[b, cid, rows])

        @pl.when(sid == 15)
        def _w_last():
            rows = pl.ds(15 * WR, WR_LAST)
            pltpu.sync_copy(acc_sh.at[rows], out_hbm.at[b, cid, rows])

    bufs = (rows_v, rows1_v, rows2_v)
    sgs = (sg0, sg1, sg2)
    sss = (ss0, ss1, ss2)

    def _wait(slot, sems):
        pltpu.make_async_copy(tabs_hbm.at[0].at[pl.ds(0, CHUNK)],
                              bufs[slot], sems[slot]).wait()

    def _issue_g(tab, j, slot):
        pltpu.async_copy(tab.at[src_v.at[j]], bufs[slot], sgs[slot])

    def _issue_s(j, slot):
        pltpu.async_copy(bufs[slot], acc_sh.at[dst_v.at[j]], sss[slot],
                         add=True)

    def _run_block(tab, hn, prefetch=None, pregathered=False):
        if not pregathered:
            _issue_g(tab, 0, 0)
            _issue_g(tab, 1, 1)
            _issue_g(tab, 2, 2)
        _wait(0, sgs)
        _issue_s(0, 0)

        def _stepj(j, slot):
            _wait(slot, sss)
            _issue_g(tab, j, slot)
            gslot = (slot + 1) % 3
            _wait(gslot, sgs)
            _issue_s(j - 2, gslot)

        ntrip = (hn - 3) // 3
        if ntrip > 0:
            def _triple(m, carry):
                j0 = 3 + 3 * m
                _stepj(j0, 0)
                _stepj(j0 + 1, 1)
                _stepj(j0 + 2, 2)
                return carry

            lax.fori_loop(0, ntrip, _triple, 0)
        for j in range(3 + 3 * ntrip, hn):
            _stepj(j, j % 3)
        _wait(hn % 3, sss)
        _wait((hn + 1) % 3, sgs)
        _issue_s(hn - 2, (hn + 1) % 3)
        _wait((hn + 1) % 3, sss)
        _wait((hn + 2) % 3, sgs)
        _issue_s(hn - 1, (hn + 2) % 3)
        if prefetch is not None:
            prefetch[0]()
        _wait((hn + 2) % 3, sss)
        if prefetch is not None:
            prefetch[1]()

    def _idx_prefetch_src(srcr, blk_i, hn):
        pltpu.async_copy(srcr.at[wid, pl.ds(blk_i * BLK, hn)],
                         src_v.at[pl.ds(0, hn)], sidx)

    def _idx_prefetch_dst(dstr, blk_i, hn):
        pltpu.async_copy(dstr.at[wid, pl.ds(blk_i * BLK, hn)],
                         dst_v.at[pl.ds(0, hn)], sidx)

    def _idx_wait(hn):
        for ref in (src_v, dst_v):
            pltpu.make_async_copy(s0_hbm.at[0, pl.ds(0, hn)],
                                  ref.at[pl.ds(0, hn)], sidx).wait()

    def _load_idx(srcr, dstr, blk_i, hn):
        pltpu.sync_copy(srcr.at[wid, pl.ds(blk_i * BLK, hn)],
                        src_v.at[pl.ds(0, hn)])
        pltpu.sync_copy(dstr.at[wid, pl.ds(blk_i * BLK, hn)],
                        dst_v.at[pl.ds(0, hn)])

    lax.fori_loop(0, CHUNK, _zrow, 0)
    _load_idx(s0_hbm, d0_hbm, 0, BLK)
    _issue_g(tabs_hbm.at[0], 1, 1)
    _issue_g(tabs_hbm.at[0], 2, 2)
    _zero_acc()
    _issue_g(tabs_hbm.at[0], 0, 0)
    plsc.subcore_barrier()

    nblocks = len(BLOCKS)
    for b, (srcr, dstr) in enumerate(((s0_hbm, d0_hbm), (s1_hbm, d1_hbm))):
        tab = tabs_hbm.at[b]
        for blk_i, hn in enumerate(BLOCKS):
            if blk_i != 0:
                _idx_wait(hn)
            if blk_i + 1 < nblocks:
                nxt = BLOCKS[blk_i + 1]
                pf = ((lambda bi=blk_i + 1, nh=nxt:
                       _idx_prefetch_src(srcr, bi, nh)),
                      (lambda bi=blk_i + 1, nh=nxt:
                       _idx_prefetch_dst(dstr, bi, nh)))
            elif b == 0:
                pf = ((lambda: _idx_prefetch_src(s1_hbm, 0, BLOCKS[0])),
                      (lambda: _idx_prefetch_dst(d1_hbm, 0, BLOCKS[0])))
            else:
                pf = None
            _run_block(tab, hn, prefetch=pf, pregathered=(blk_i == 0))
        plsc.subcore_barrier()

        if b == 0:
            _idx_wait(BLK)
            _issue_g(tabs_hbm.at[1], 1, 1)
            _issue_g(tabs_hbm.at[1], 2, 2)
            _write_out(0)
            lax.fori_loop(0, CHUNK, _zrow, 0)
            _zero_acc()
            _issue_g(tabs_hbm.at[1], 0, 0)
            plsc.subcore_barrier()
        else:
            _write_out(1)


def _segment_sum2(tables, edges0, edges1):
    d0 = edges0[0].reshape(NW, NCH, CHUNK)
    s0 = edges0[1].reshape(NW, NCH, CHUNK)
    d1 = edges1[0].reshape(NW, NCH, CHUNK)
    s1 = edges1[1].reshape(NW, NCH, CHUNK)
    return _segsum_sc(tables, s0, d0, s1, d1)


RB = 2000
NB = N // RB


def _leaky(x):
    return jnp.where(x > 0, x, ALPHA_SLOPE * x)


def _mm_batched_body(x_ref, w_ref, o_ref):
    o_ref[...] = jnp.dot(x_ref[0], w_ref[0],
                         preferred_element_type=jnp.float32)[None]


def _support1(ufeas, gw1):
    return pl.pallas_call(
        _mm_batched_body,
        grid=(2, NB),
        in_specs=[
            pl.BlockSpec((1, RB, D), lambda i, b: (i, b, 0)),
            pl.BlockSpec((1, D, D), lambda i, b: (i, 0, 0)),
        ],
        out_specs=pl.BlockSpec((1, RB, D), lambda i, b: (i, b, 0)),
        out_shape=jax.ShapeDtypeStruct((2, N, D), jnp.float32),
    )(ufeas, gw1)


def _stage_mid_body(p_ref, b_ref, w_ref, o_ref):
    agg = p_ref[0, 0] + p_ref[0, 1]
    h = _leaky(agg + b_ref[0, 0])
    o_ref[...] = jnp.dot(h, w_ref[0], preferred_element_type=jnp.float32)[None]


def _stage_mid(parts, bias, w):
    return pl.pallas_call(
        _stage_mid_body,
        grid=(2, NB),
        in_specs=[
            pl.BlockSpec((1, 2, RB, D), lambda i, bk: (i, 0, bk, 0)),
            pl.BlockSpec((1, 1, D), lambda i, bk: (i, 0, 0)),
            pl.BlockSpec((1, D, D), lambda i, bk: (i, 0, 0)),
        ],
        out_specs=pl.BlockSpec((1, RB, D), lambda i, bk: (i, bk, 0)),
        out_shape=jax.ShapeDtypeStruct((2, N, D), jnp.float32),
    )(parts, bias.reshape(2, 1, D), w)


def _head_body(p_ref, gb_ref, uf_ref, wa_ref, wb_ref, ub_ref, o_ref):
    acc = None
    for i in range(2):
        h = _leaky(p_ref[i, 0] + p_ref[i, 1] + gb_ref[i, 0])
        out = (jnp.dot(h, wa_ref[i], preferred_element_type=jnp.float32)
               + jnp.dot(uf_ref[i], wb_ref[i],
                         preferred_element_type=jnp.float32)
               + ub_ref[i, 0])
        r = jnp.maximum(out, 0.0)
        acc = RATE_MIX * r if acc is None else acc + (1.0 - RATE_MIX) * r
    o_ref[...] = acc


def _head(parts, gb2, ufeas, uwa, uwb, ub):
    return pl.pallas_call(
        _head_body,
        grid=(NB,),
        in_specs=[
            pl.BlockSpec((2, 2, RB, D), lambda bk: (0, 0, bk, 0)),
            pl.BlockSpec((2, 1, D), lambda bk: (0, 0, 0)),
            pl.BlockSpec((2, RB, D), lambda bk: (0, bk, 0)),
            pl.BlockSpec((2, D, D), lambda bk: (0, 0, 0)),
            pl.BlockSpec((2, D, D), lambda bk: (0, 0, 0)),
            pl.BlockSpec((2, 1, D), lambda bk: (0, 0, 0)),
        ],
        out_specs=pl.BlockSpec((RB, D), lambda bk: (bk, 0)),
        out_shape=jax.ShapeDtypeStruct((N, D), jnp.float32),
    )(parts, gb2.reshape(2, 1, D), ufeas, uwa, uwb, ub.reshape(2, 1, D))


def kernel(UFEAs, UVs, VUs, gw1, gb1, gw2, gb2, uw, ub):
    support1 = _support1(UFEAs, gw1)
    p1 = _segment_sum2(support1, VUs[0], VUs[1])
    support2 = _stage_mid(p1, gb1, gw2)
    p2 = _segment_sum2(support2, UVs[0], UVs[1])
    return _head(p2, gb2, UFEAs, uw[:, :D], uw[:, D:], ub)


# speedup vs baseline: 1.2261x; 1.0177x over previous
import functools

import jax
import jax.numpy as jnp
from jax import lax
from jax.experimental import pallas as pl
from jax.experimental.pallas import tpu as pltpu
from jax.experimental.pallas import tpu_sc as plsc

N = 10000
E = 320000
D = 128
ALPHA_SLOPE = 0.2
RATE_MIX = 0.5

NW = 32
CHUNK = 80
NCH = 125
BLK = 40
BLOCKS = (BLK, BLK, NCH - 2 * BLK)
BLK_MAX = max(BLOCKS)
WR = 624
WR_LAST = N - 15 * WR

_MESH = plsc.VectorSubcoreMesh(core_axis_name="c", subcore_axis_name="s")


@functools.partial(
    pl.kernel,
    mesh=_MESH,
    out_type=jax.ShapeDtypeStruct((2, 2, N, D), jnp.float32),
    scratch_types=[
        pltpu.VMEM((BLK_MAX, CHUNK), jnp.int32),
        pltpu.VMEM((BLK_MAX, CHUNK), jnp.int32),
        pltpu.VMEM((CHUNK, D), jnp.float32),
        pltpu.VMEM((CHUNK, D), jnp.float32),
        pltpu.VMEM((CHUNK, D), jnp.float32),
        pltpu.VMEM_SHARED((N, D), jnp.float32),
        pltpu.SemaphoreType.DMA,
        pltpu.SemaphoreType.DMA,
        pltpu.SemaphoreType.DMA,
        pltpu.SemaphoreType.DMA,
        pltpu.SemaphoreType.DMA,
        pltpu.SemaphoreType.DMA,
        pltpu.SemaphoreType.DMA,
    ],
)
def _segsum_sc(tabs_hbm, s0_hbm, d0_hbm, s1_hbm, d1_hbm, out_hbm,
               src_v, dst_v, rows_v, rows1_v, rows2_v, acc_sh,
               sg0, sg1, sg2, ss0, ss1, ss2, sidx):
    cid = lax.axis_index("c")
    sid = lax.axis_index("s")
    wid = sid * 2 + cid

    zvec = jnp.zeros((16,), jnp.float32)

    def _zrow(r, carry):
        for k in range(D // 16):
            rows_v[r, pl.ds(k * 16, 16)] = zvec
        return carry

    def _zero_acc():
        base = sid * WR

        @pl.when(sid < 15)
        def _z_main():
            for t in range(WR // CHUNK):
                pltpu.sync_copy(rows_v,
                                acc_sh.at[pl.ds(base + t * CHUNK, CHUNK)])
            pltpu.sync_copy(rows_v.at[pl.ds(0, WR % CHUNK)],
                            acc_sh.at[pl.ds(base + WR - WR % CHUNK,
                                            WR % CHUNK)])

        @pl.when(sid == 15)
        def _z_last():
            for t in range(WR_LAST // CHUNK):
                pltpu.sync_copy(rows_v,
                                acc_sh.at[pl.ds(base + t * CHUNK, CHUNK)])

    def _write_out(b):
        @pl.when(sid < 15)
        def _w_main():
            rows = pl.ds(sid * WR, WR)
            pltpu.sync_copy(acc_sh.at[rows], out_hbm.at[b, cid, rows])

        @pl.when(sid == 15)
        def _w_last():
            rows = pl.ds(15 * WR, WR_LAST)
            pltpu.sync_copy(acc_sh.at[rows], out_hbm.at[b, cid, rows])

    bufs = (rows_v, rows1_v, rows2_v)
    sgs = (sg0, sg1, sg2)
    sss = (ss0, ss1, ss2)

    def _wait(slot, sems):
        pltpu.make_async_copy(tabs_hbm.at[0].at[pl.ds(0, CHUNK)],
                              bufs[slot], sems[slot]).wait()

    def _issue_g(tab, j, slot):
        pltpu.async_copy(tab.at[src_v.at[j]], bufs[slot], sgs[slot])

    def _issue_s(j, slot):
        pltpu.async_copy(bufs[slot], acc_sh.at[dst_v.at[j]], sss[slot],
                         add=True)

    def _run_block(tab, hn, prefetch=None, pregathered=False):
        if not pregathered:
            _issue_g(tab, 0, 0)
            _issue_g(tab, 1, 1)
            _issue_g(tab, 2, 2)
        _wait(0, sgs)
        _issue_s(0, 0)

        def _stepj(j, slot):
            _wait(slot, sss)
            _issue_g(tab, j, slot)
            gslot = (slot + 1) % 3
            _wait(gslot, sgs)
            _issue_s(j - 2, gslot)

        ntrip = (hn - 3) // 3
        if ntrip > 0:
            def _triple(m, carry):
                j0 = 3 + 3 * m
                _stepj(j0, 0)
                _stepj(j0 + 1, 1)
                _stepj(j0 + 2, 2)
                return carry

            lax.fori_loop(0, ntrip, _triple, 0)
        for j in range(3 + 3 * ntrip, hn):
            _stepj(j, j % 3)
        _wait(hn % 3, sss)
        _wait((hn + 1) % 3, sgs)
        _issue_s(hn - 2, (hn + 1) % 3)
        _wait((hn + 1) % 3, sss)
        _wait((hn + 2) % 3, sgs)
        _issue_s(hn - 1, (hn + 2) % 3)
        if prefetch is not None:
            prefetch[0]()
        _wait((hn + 2) % 3, sss)
        if prefetch is not None:
            prefetch[1]()

    def _idx_parts(hn):
        return ((0, BLK), (BLK, hn - BLK)) if hn > BLK else ((0, hn),)

    def _idx_prefetch_src(srcr, blk_i, hn):
        for off, ln in _idx_parts(hn):
            pltpu.async_copy(srcr.at[wid, pl.ds(blk_i * BLK + off, ln)],
                             src_v.at[pl.ds(off, ln)], sidx)

    def _idx_prefetch_dst(dstr, blk_i, hn):
        for off, ln in _idx_parts(hn):
            pltpu.async_copy(dstr.at[wid, pl.ds(blk_i * BLK + off, ln)],
                             dst_v.at[pl.ds(off, ln)], sidx)

    def _idx_wait(hn):
        for ref in (src_v, dst_v):
            for off, ln in _idx_parts(hn):
                pltpu.make_async_copy(s0_hbm.at[0, pl.ds(0, ln)],
                                      ref.at[pl.ds(off, ln)], sidx).wait()

    def _load_idx(srcr, dstr, blk_i, hn):
        pltpu.sync_copy(srcr.at[wid, pl.ds(blk_i * BLK, hn)],
                        src_v.at[pl.ds(0, hn)])
        pltpu.sync_copy(dstr.at[wid, pl.ds(blk_i * BLK, hn)],
                        dst_v.at[pl.ds(0, hn)])

    lax.fori_loop(0, CHUNK, _zrow, 0)
    _load_idx(s0_hbm, d0_hbm, 0, BLK)
    _issue_g(tabs_hbm.at[0], 1, 1)
    _issue_g(tabs_hbm.at[0], 2, 2)
    _zero_acc()
    _issue_g(tabs_hbm.at[0], 0, 0)
    plsc.subcore_barrier()

    nblocks = len(BLOCKS)
    for b, (srcr, dstr) in enumerate(((s0_hbm, d0_hbm), (s1_hbm, d1_hbm))):
        tab = tabs_hbm.at[b]
        for blk_i, hn in enumerate(BLOCKS):
            if blk_i != 0:
                _idx_wait(hn)
            if blk_i + 1 < nblocks:
                nxt = BLOCKS[blk_i + 1]
                pf = ((lambda bi=blk_i + 1, nh=nxt:
                       _idx_prefetch_src(srcr, bi, nh)),
                      (lambda bi=blk_i + 1, nh=nxt:
                       _idx_prefetch_dst(dstr, bi, nh)))
            elif b == 0:
                pf = ((lambda: _idx_prefetch_src(s1_hbm, 0, BLOCKS[0])),
                      (lambda: _idx_prefetch_dst(d1_hbm, 0, BLOCKS[0])))
            else:
                pf = None
            _run_block(tab, hn, prefetch=pf, pregathered=(blk_i == 0))
        plsc.subcore_barrier()

        if b == 0:
            _idx_wait(BLK)
            _issue_g(tabs_hbm.at[1], 1, 1)
            _issue_g(tabs_hbm.at[1], 2, 2)
            _write_out(0)
            lax.fori_loop(0, CHUNK, _zrow, 0)
            _zero_acc()
            _issue_g(tabs_hbm.at[1], 0, 0)
            plsc.subcore_barrier()
        else:
            _write_out(1)


def _segment_sum2(tables, edges0, edges1):
    d0 = edges0[0].reshape(NW, NCH, CHUNK)
    s0 = edges0[1].reshape(NW, NCH, CHUNK)
    d1 = edges1[0].reshape(NW, NCH, CHUNK)
    s1 = edges1[1].reshape(NW, NCH, CHUNK)
    return _segsum_sc(tables, s0, d0, s1, d1)


RB = 2000
NB = N // RB


def _leaky(x):
    return jnp.where(x > 0, x, ALPHA_SLOPE * x)


def _mm_batched_body(x_ref, w_ref, o_ref):
    o_ref[...] = jnp.dot(x_ref[0], w_ref[0],
                         preferred_element_type=jnp.float32)[None]


def _support1(ufeas, gw1):
    return pl.pallas_call(
        _mm_batched_body,
        grid=(2, NB),
        in_specs=[
            pl.BlockSpec((1, RB, D), lambda i, b: (i, b, 0)),
            pl.BlockSpec((1, D, D), lambda i, b: (i, 0, 0)),
        ],
        out_specs=pl.BlockSpec((1, RB, D), lambda i, b: (i, b, 0)),
        out_shape=jax.ShapeDtypeStruct((2, N, D), jnp.float32),
    )(ufeas, gw1)


def _stage_mid_body(p_ref, b_ref, w_ref, o_ref):
    agg = p_ref[0, 0] + p_ref[0, 1]
    h = _leaky(agg + b_ref[0, 0])
    o_ref[...] = jnp.dot(h, w_ref[0], preferred_element_type=jnp.float32)[None]


def _stage_mid(parts, bias, w):
    return pl.pallas_call(
        _stage_mid_body,
        grid=(2, NB),
        in_specs=[
            pl.BlockSpec((1, 2, RB, D), lambda i, bk: (i, 0, bk, 0)),
            pl.BlockSpec((1, 1, D), lambda i, bk: (i, 0, 0)),
            pl.BlockSpec((1, D, D), lambda i, bk: (i, 0, 0)),
        ],
        out_specs=pl.BlockSpec((1, RB, D), lambda i, bk: (i, bk, 0)),
        out_shape=jax.ShapeDtypeStruct((2, N, D), jnp.float32),
    )(parts, bias.reshape(2, 1, D), w)


def _head_body(p_ref, gb_ref, uf_ref, wa_ref, wb_ref, ub_ref, o_ref):
    acc = None
    for i in range(2):
        h = _leaky(p_ref[i, 0] + p_ref[i, 1] + gb_ref[i, 0])
        out = (jnp.dot(h, wa_ref[i], preferred_element_type=jnp.float32)
               + jnp.dot(uf_ref[i], wb_ref[i],
                         preferred_element_type=jnp.float32)
               + ub_ref[i, 0])
        r = jnp.maximum(out, 0.0)
        acc = RATE_MIX * r if acc is None else acc + (1.0 - RATE_MIX) * r
    o_ref[...] = acc


def _head(parts, gb2, ufeas, uwa, uwb, ub):
    return pl.pallas_call(
        _head_body,
        grid=(NB,),
        in_specs=[
            pl.BlockSpec((2, 2, RB, D), lambda bk: (0, 0, bk, 0)),
            pl.BlockSpec((2, 1, D), lambda bk: (0, 0, 0)),
            pl.BlockSpec((2, RB, D), lambda bk: (0, bk, 0)),
            pl.BlockSpec((2, D, D), lambda bk: (0, 0, 0)),
            pl.BlockSpec((2, D, D), lambda bk: (0, 0, 0)),
            pl.BlockSpec((2, 1, D), lambda bk: (0, 0, 0)),
        ],
        out_specs=pl.BlockSpec((RB, D), lambda bk: (bk, 0)),
        out_shape=jax.ShapeDtypeStruct((N, D), jnp.float32),
    )(parts, gb2.reshape(2, 1, D), ufeas, uwa, uwb, ub.reshape(2, 1, D))


def kernel(UFEAs, UVs, VUs, gw1, gb1, gw2, gb2, uw, ub):
    support1 = _support1(UFEAs, gw1)
    p1 = _segment_sum2(support1, VUs[0], VUs[1])
    support2 = _stage_mid(p1, gb1, gw2)
    p2 = _segment_sum2(support2, UVs[0], UVs[1])
    return _head(p2, gb2, UFEAs, uw[:, :D], uw[:, D:], ub)
